# Initial kernel scaffold; baseline (speedup 1.0000x reference)
import jax
import jax.numpy as jnp
from jax.experimental import pallas as pl


def kernel(target, context, negatives, target_table, context_table):
    raise NotImplementedError("write your pallas kernel here")


# speedup vs baseline: 3.4077x; 3.4077x over previous
import functools

import jax
import jax.numpy as jnp
from jax import lax
from jax.experimental import pallas as pl
from jax.experimental.pallas import tpu as pltpu
from jax.experimental.pallas import tpu_sc as plsc

VOCAB = 1000000
DIM = 64
BATCH = 16384
N_NEG = 20

NC, NS = 2, 16
NW = NC * NS

CHUNK = 128

BC_PER_W = BATCH // NW
NEG_PER_W = BATCH * N_NEG // NW


def _gather_rows(idx_hbm, table_hbm, out_hbm, base, n_chunks, idx_v, rows_v, sem):
    def body(k, _):
        off = base + k * CHUNK
        pltpu.sync_copy(idx_hbm.at[pl.ds(off, CHUNK)], idx_v)
        pltpu.async_copy(table_hbm.at[idx_v], rows_v, sem).wait()
        pltpu.sync_copy(rows_v, out_hbm.at[pl.ds(off, CHUNK)])
        return 0

    lax.fori_loop(0, n_chunks, body, 0)


def _sc_gather_all(target, context, neg_flat, target_table, context_table):
    mesh = plsc.VectorSubcoreMesh(core_axis_name="c", subcore_axis_name="s")

    @functools.partial(
        pl.kernel,
        out_type=(
            jax.ShapeDtypeStruct((BATCH, DIM), jnp.float32),
            jax.ShapeDtypeStruct((BATCH, DIM), jnp.float32),
            jax.ShapeDtypeStruct((BATCH * N_NEG, DIM), jnp.float32),
        ),
        mesh=mesh,
        compiler_params=pltpu.CompilerParams(use_tc_tiling_on_sc=False),
        scratch_types=[
            pltpu.VMEM((CHUNK,), jnp.int32),
            pltpu.VMEM((CHUNK, DIM), jnp.float32),
            pltpu.SemaphoreType.DMA,
        ],
    )
    def k(tgt_hbm, ctx_hbm, neg_hbm, ttab_hbm, ctab_hbm,
          t_out, c_out, n_out, idx_v, rows_v, sem):
        wid = lax.axis_index("s") * NC + lax.axis_index("c")
        _gather_rows(tgt_hbm, ttab_hbm, t_out, wid * BC_PER_W,
                     BC_PER_W // CHUNK, idx_v, rows_v, sem)
        _gather_rows(ctx_hbm, ctab_hbm, c_out, wid * BC_PER_W,
                     BC_PER_W // CHUNK, idx_v, rows_v, sem)
        _gather_rows(neg_hbm, ctab_hbm, n_out, wid * NEG_PER_W,
                     NEG_PER_W // CHUNK, idx_v, rows_v, sem)

    return k(target, context, neg_flat, target_table, context_table)


BLK = 512
NB = BATCH // BLK


def _tc_loss_body(t_ref, c_ref, n_ref, out_ref):
    b = pl.program_id(0)
    k = pl.program_id(1)
    t = t_ref[...]
    other = jnp.where(k == 0, c_ref[...], n_ref[...])
    sgn = jnp.where(k == 0, 1.0, -1.0).astype(jnp.float32)
    s = jnp.sum(t * other, axis=1, keepdims=True)
    val = jnp.sum(jnp.log(jax.nn.sigmoid(sgn * s) + 1e-10))
    first = jnp.logical_and(b == 0, k == 0)
    last = jnp.logical_and(b == NB - 1, k == N_NEG)
    acc = jnp.where(first, jnp.zeros((1, 1), jnp.float32), out_ref[...]) + val
    out_ref[...] = jnp.where(last, -acc / BATCH, acc)


def _tc_loss(t_emb, c_emb, n_emb):
    return pl.pallas_call(
        _tc_loss_body,
        grid=(NB, N_NEG + 1),
        in_specs=[
            pl.BlockSpec((BLK, DIM), lambda b, k: (b, 0)),
            pl.BlockSpec((BLK, DIM), lambda b, k: (b, 0)),
            pl.BlockSpec((BLK, DIM),
                         lambda b, k: (jnp.maximum(k - 1, 0) * NB + b, 0)),
        ],
        out_specs=pl.BlockSpec((1, 1), lambda b, k: (0, 0)),
        out_shape=jax.ShapeDtypeStruct((1, 1), jnp.float32),
    )(t_emb, c_emb, n_emb)


def kernel(target, context, negatives, target_table, context_table):
    target = target.astype(jnp.int32)
    context = context.astype(jnp.int32)
    neg_flat = negatives.astype(jnp.int32).T.reshape(-1)
    t_emb, c_emb, n_emb = _sc_gather_all(
        target, context, neg_flat, target_table, context_table)
    loss = _tc_loss(t_emb, c_emb, n_emb)
    return loss[0, 0]


# speedup vs baseline: 4.5643x; 1.3394x over previous
import functools

import jax
import jax.numpy as jnp
from jax import lax
from jax.experimental import pallas as pl
from jax.experimental.pallas import tpu as pltpu
from jax.experimental.pallas import tpu_sc as plsc

VOCAB = 1000000
DIM = 64
BATCH = 16384
N_NEG = 20

NC, NS = 2, 16
NW = NC * NS

CHUNK = 512
BC_PER_W = BATCH // NW
NEG_PER_W = BATCH * N_NEG // NW
NEG_CHUNKS = NEG_PER_W // CHUNK


def _sc_gather_all(target, context, neg_flat, target_table, context_table):
    mesh = plsc.VectorSubcoreMesh(core_axis_name="c", subcore_axis_name="s")

    @functools.partial(
        pl.kernel,
        out_type=(
            jax.ShapeDtypeStruct((BATCH, DIM), jnp.float32),
            jax.ShapeDtypeStruct((BATCH, DIM), jnp.float32),
            jax.ShapeDtypeStruct((BATCH * N_NEG, DIM), jnp.float32),
        ),
        mesh=mesh,
        compiler_params=pltpu.CompilerParams(use_tc_tiling_on_sc=False),
        scratch_types=[
            pltpu.VMEM((2 * BC_PER_W,), jnp.int32),
            pltpu.VMEM((NEG_PER_W,), jnp.int32),
            pltpu.VMEM((3, CHUNK, DIM), jnp.float32),
            pltpu.SemaphoreType.DMA,
            pltpu.SemaphoreType.DMA,
            pltpu.SemaphoreType.DMA,
            pltpu.SemaphoreType.DMA,
            pltpu.SemaphoreType.DMA,
            pltpu.SemaphoreType.DMA,
        ],
    )
    def k(tgt_hbm, ctx_hbm, neg_hbm, ttab_hbm, ctab_hbm,
          t_out, c_out, n_out, tci_v, negidx_v, rows_v,
          g0, g1, g2, s0, s1, s2):
        g = (g0, g1, g2)
        s = (s0, s1, s2)
        wid = lax.axis_index("s") * NC + lax.axis_index("c")
        base_tc = wid * BC_PER_W
        base_n = wid * NEG_PER_W

        pltpu.sync_copy(tgt_hbm.at[pl.ds(base_tc, BC_PER_W)],
                        tci_v.at[pl.ds(0, BC_PER_W)])
        pltpu.sync_copy(ctx_hbm.at[pl.ds(base_tc, BC_PER_W)],
                        tci_v.at[pl.ds(BC_PER_W, BC_PER_W)])
        gh = {}
        sh = {}
        gh[0] = pltpu.async_copy(
            ttab_hbm.at[tci_v.at[pl.ds(0, BC_PER_W)]], rows_v.at[0], g0)
        gh[1] = pltpu.async_copy(
            ctab_hbm.at[tci_v.at[pl.ds(BC_PER_W, BC_PER_W)]], rows_v.at[1], g1)
        pltpu.sync_copy(neg_hbm.at[pl.ds(base_n, NEG_PER_W)], negidx_v)
        gh[0].wait()
        sh["t"] = pltpu.async_copy(
            rows_v.at[0], t_out.at[pl.ds(base_tc, BC_PER_W)], s0)
        gh[1].wait()
        sh["c"] = pltpu.async_copy(
            rows_v.at[1], c_out.at[pl.ds(base_tc, BC_PER_W)], s1)

        for kk in range(NEG_CHUNKS):
            b = kk % 3
            if kk == 0:
                sh["t"].wait()
            elif kk == 1:
                sh["c"].wait()
            elif kk >= 3:
                sh[kk - 3].wait()
            gh[kk] = pltpu.async_copy(
                ctab_hbm.at[negidx_v.at[pl.ds(kk * CHUNK, CHUNK)]],
                rows_v.at[b], g[b])
            if kk >= 2:
                j = kk - 2
                gh[j].wait()
                sh[j] = pltpu.async_copy(
                    rows_v.at[j % 3],
                    n_out.at[pl.ds(base_n + j * CHUNK, CHUNK)], s[j % 3])
        for j in (NEG_CHUNKS - 2, NEG_CHUNKS - 1):
            gh[j].wait()
            sh[j] = pltpu.async_copy(
                rows_v.at[j % 3],
                n_out.at[pl.ds(base_n + j * CHUNK, CHUNK)], s[j % 3])
        for j in (NEG_CHUNKS - 3, NEG_CHUNKS - 2, NEG_CHUNKS - 1):
            sh[j].wait()

    return k(target, context, neg_flat, target_table, context_table)


BLK = 512
NB = BATCH // BLK


def _tc_loss_body(t_ref, c_ref, n_ref, out_ref):
    b = pl.program_id(0)
    t = t_ref[...]
    pos = jnp.sum(t * c_ref[...], axis=1, keepdims=True)
    val = jnp.sum(jnp.log(jax.nn.sigmoid(pos) + 1e-10))
    ns = jnp.sum(n_ref[...] * t[None, :, :], axis=2)
    val += jnp.sum(jnp.log(jax.nn.sigmoid(-ns) + 1e-10))
    acc = jnp.where(b == 0, jnp.zeros((1, 1), jnp.float32), out_ref[...]) + val
    out_ref[...] = jnp.where(b == NB - 1, -acc / BATCH, acc)


def _tc_loss(t_emb, c_emb, n_emb3):
    return pl.pallas_call(
        _tc_loss_body,
        grid=(NB,),
        in_specs=[
            pl.BlockSpec((BLK, DIM), lambda b: (b, 0)),
            pl.BlockSpec((BLK, DIM), lambda b: (b, 0)),
            pl.BlockSpec((N_NEG, BLK, DIM), lambda b: (0, b, 0)),
        ],
        out_specs=pl.BlockSpec((1, 1), lambda b: (0, 0)),
        out_shape=jax.ShapeDtypeStruct((1, 1), jnp.float32),
    )(t_emb, c_emb, n_emb3)


def kernel(target, context, negatives, target_table, context_table):
    target = target.astype(jnp.int32)
    context = context.astype(jnp.int32)
    neg_flat = negatives.astype(jnp.int32).T.reshape(-1)
    t_emb, c_emb, n_emb = _sc_gather_all(
        target, context, neg_flat, target_table, context_table)
    loss = _tc_loss(t_emb, c_emb, n_emb.reshape(N_NEG, BATCH, DIM))
    return loss[0, 0]


# speedup vs baseline: 5.4638x; 1.1971x over previous
import functools

import jax
import jax.numpy as jnp
from jax import lax
from jax.experimental import pallas as pl
from jax.experimental.pallas import tpu as pltpu
from jax.experimental.pallas import tpu_sc as plsc

VOCAB = 1000000
DIM = 64
BATCH = 16384
N_NEG = 20
NPAD = 32

NC, NS = 2, 16
NW = NC * NS

BC_PER_W = BATCH // NW
NEG_PER_W = BATCH * N_NEG // NW

CHUNK = 160
BG_PER_CHUNK = CHUNK // N_NEG
NCH = NEG_PER_W // CHUNK
NBUF = 4

C_CHUNKS = (160, 160, 160, 32)
C_OFFS = (0, 160, 320, 480)


def _dot(rows_v, r, row, t_rows, gb):
    acc = None
    for q in range(4):
        nv = rows_v[r, row, pl.ds(q * 16, 16)]
        tv = t_rows[gb, pl.ds(q * 16, 16)]
        acc = nv * tv if acc is None else acc + nv * tv
    return jnp.sum(acc)


def _sc_scores(target, context, neg_flat, target_table, context_table):
    mesh = plsc.VectorSubcoreMesh(core_axis_name="c", subcore_axis_name="s")

    @functools.partial(
        pl.kernel,
        out_type=(
            jax.ShapeDtypeStruct((BATCH,), jnp.float32),
            jax.ShapeDtypeStruct((BATCH, NPAD), jnp.float32),
        ),
        mesh=mesh,
        compiler_params=pltpu.CompilerParams(use_tc_tiling_on_sc=False,
                                             needs_layout_passes=False),
        scratch_types=[
            pltpu.VMEM((2 * BC_PER_W,), jnp.int32),
            pltpu.VMEM((NEG_PER_W,), jnp.int32),
            pltpu.VMEM((NBUF, CHUNK, DIM), jnp.float32),
            pltpu.VMEM((BC_PER_W, DIM), jnp.float32),
            pltpu.VMEM((BC_PER_W,), jnp.float32),
            pltpu.VMEM((BC_PER_W, NPAD), jnp.float32),
            pltpu.SemaphoreType.DMA,
            pltpu.SemaphoreType.DMA,
            pltpu.SemaphoreType.DMA,
            pltpu.SemaphoreType.DMA,
            pltpu.SemaphoreType.DMA,
        ],
    )
    def k(tgt_hbm, ctx_hbm, neg_hbm, ttab_hbm, ctab_hbm, pos_out, neg_out,
          tci_v, negidx_v, rows_v, t_rows, pos_v, negs_v, tg, g0, g1, g2, g3):
        g = (g0, g1, g2, g3)
        lane = lax.iota(jnp.int32, 16)
        wid = lax.axis_index("s") * NC + lax.axis_index("c")
        base_tc = wid * BC_PER_W
        base_n = wid * NEG_PER_W

        pltpu.sync_copy(tgt_hbm.at[pl.ds(base_tc, BC_PER_W)],
                        tci_v.at[pl.ds(0, BC_PER_W)])
        pltpu.sync_copy(ctx_hbm.at[pl.ds(base_tc, BC_PER_W)],
                        tci_v.at[pl.ds(BC_PER_W, BC_PER_W)])
        pltpu.sync_copy(neg_hbm.at[pl.ds(base_n, NEG_PER_W)], negidx_v)

        th = pltpu.async_copy(
            ttab_hbm.at[tci_v.at[pl.ds(0, BC_PER_W)]], t_rows, tg)

        ch = []
        for r in range(NBUF):
            ch.append(pltpu.async_copy(
                ctab_hbm.at[tci_v.at[pl.ds(BC_PER_W + C_OFFS[r], C_CHUNKS[r])]],
                rows_v.at[r, pl.ds(0, C_CHUNKS[r])], g[r]))
        th.wait()

        for r in range(NBUF):
            ch[r].wait()
            coff = C_OFFS[r]

            def pos_body(pg, _, r=r, coff=coff):
                pvec = jnp.zeros((16,), jnp.float32)
                for jj in range(16):
                    row = pg * 16 + jj
                    s = _dot(rows_v, r, row, t_rows, coff + row)
                    pvec = jnp.where(lane == jj, s, pvec)
                pos_v[pl.ds(coff + pg * 16, 16)] = pvec
                return 0

            lax.fori_loop(0, C_CHUNKS[r] // 16, pos_body, 0)
            pltpu.async_copy(
                ctab_hbm.at[negidx_v.at[pl.ds(r * CHUNK, CHUNK)]],
                rows_v.at[r], g[r])

        def neg_iter(i, _):
            for r in range(NBUF):
                kk = i * NBUF + r
                pltpu.make_async_copy(
                    ctab_hbm.at[pl.ds(0, CHUNK)], rows_v.at[r], g[r]).wait()

                def neg_body(g8, _, r=r):
                    gb = kk * BG_PER_CHUNK + g8
                    nv0 = jnp.zeros((16,), jnp.float32)
                    nv1 = jnp.zeros((16,), jnp.float32)
                    for n in range(N_NEG):
                        s = _dot(rows_v, r, g8 * N_NEG + n, t_rows, gb)
                        if n < 16:
                            nv0 = jnp.where(lane == n, s, nv0)
                        else:
                            nv1 = jnp.where(lane == n - 16, s, nv1)
                    negs_v[gb, pl.ds(0, 16)] = nv0
                    negs_v[gb, pl.ds(16, 16)] = nv1
                    return 0

                lax.fori_loop(0, BG_PER_CHUNK, neg_body, 0)

                @pl.when(kk + NBUF < NCH)
                def _():
                    pltpu.async_copy(
                        ctab_hbm.at[negidx_v.at[pl.ds((kk + NBUF) * CHUNK,
                                                      CHUNK)]],
                        rows_v.at[r], g[r])
            return 0

        lax.fori_loop(0, NCH // NBUF, neg_iter, 0)
        pltpu.sync_copy(pos_v, pos_out.at[pl.ds(base_tc, BC_PER_W)])
        pltpu.sync_copy(negs_v, neg_out.at[pl.ds(base_tc, BC_PER_W)])

    return k(target, context, neg_flat, target_table, context_table)


PROWS = BATCH // 128
NROWS = BATCH * NPAD // 128


def _tc_loss_body(p_ref, n_ref, out_ref):
    p = p_ref[...]
    val = jnp.sum(jnp.log(jax.nn.sigmoid(p) + 1e-10))
    x = n_ref[...]
    c_io = lax.broadcasted_iota(jnp.int32, (NROWS, 128), 1)
    valid = (c_io % NPAD) < N_NEG
    xs = jnp.where(valid, x, 0.0)
    nl = jnp.log(jax.nn.sigmoid(-xs) + 1e-10)
    val += jnp.sum(jnp.where(valid, nl, 0.0))
    out_ref[...] = jnp.full((1, 1), -1.0 / BATCH, jnp.float32) * val


def _tc_loss(pos2, neg2):
    return pl.pallas_call(
        _tc_loss_body,
        out_shape=jax.ShapeDtypeStruct((1, 1), jnp.float32),
    )(pos2, neg2)


def kernel(target, context, negatives, target_table, context_table):
    target = target.astype(jnp.int32)
    context = context.astype(jnp.int32)
    neg_flat = negatives.astype(jnp.int32).reshape(-1)
    pos, neg = _sc_scores(target, context, neg_flat,
                          target_table, context_table)
    loss = _tc_loss(pos.reshape(PROWS, 128), neg.reshape(NROWS, 128))
    return loss[0, 0]
